# baseline (speedup 1.0000x reference)
import functools

import jax
import jax.numpy as jnp
from jax import lax
from jax.experimental import pallas as pl
from jax.experimental.pallas import tpu as pltpu
from jax.experimental.pallas import tpu_sc as plsc


def _node_body(x_ref, wnt_ref, bn_ref, a_ref, ne_ref, tn_ref):
    h = jnp.dot(x_ref[...], wnt_ref[...], preferred_element_type=jnp.float32)
    h = jnp.maximum(h + bn_ref[...], 0.0)
    ne_ref[...] = h
    tn_ref[...] = jnp.dot(h, a_ref[...], preferred_element_type=jnp.float32)


def _edge_body(e_ref, wet_ref, be_ref, bmat_ref, bc_ref, te_ref):
    h = jnp.dot(e_ref[...], wet_ref[...], preferred_element_type=jnp.float32)
    h = jnp.maximum(h + be_ref[...], 0.0)
    t = jnp.dot(h, bmat_ref[...], preferred_element_type=jnp.float32) + bc_ref[...]
    te_ref[...] = t.reshape(te_ref.shape)


def _final_body(ne_ref, p0_ref, p1_ref, p2_ref, p3_ref, w_ref, b_ref, z_ref):
    h = (ne_ref[...] + p0_ref[...] + p1_ref[...] + p2_ref[...] + p3_ref[...])
    z = jnp.sum(h * w_ref[...], axis=1, keepdims=True) + b_ref[...]
    z_ref[...] = jax.nn.sigmoid(z)


_NC = 2
_NS = 16
_NW = _NC * _NS
_CB = 128
_GRP = 8


def _chunk_split(total_chunks):
    base_c = total_chunks // _NW
    extra = total_chunks - base_c * _NW
    max_c = base_c + (1 if extra else 0)
    groups = base_c // _GRP
    tail = base_c - groups * _GRP
    return base_c, extra, max_c, groups, tail


def _make_mesh():
    return plsc.VectorSubcoreMesh(core_axis_name="c", subcore_axis_name="s",
                                  num_cores=_NC, num_subcores=_NS)


def _make_tn_scatter_kernel(n_pad, total_chunks):
    base_c, extra, max_c, groups, tail = _chunk_split(total_chunks)
    rows_per_s = n_pad // _NS

    @functools.partial(
        pl.kernel,
        out_type=jax.ShapeDtypeStruct((_NC, n_pad, 32), jnp.float32),
        mesh=_make_mesh(),
        scratch_types=[
            pltpu.VMEM((max_c, _CB), jnp.int32),
            pltpu.VMEM((max_c, _CB), jnp.int32),
            pltpu.VMEM((_GRP, _CB, 32), jnp.float32),
            pltpu.VMEM_SHARED((n_pad, 32), jnp.float32),
            pltpu.SemaphoreType.DMA,
            pltpu.SemaphoreType.DMA,
        ],
        compiler_params=pltpu.CompilerParams(use_tc_tiling_on_sc=False,
                                             disable_bounds_checks=True),
    )
    def scatter_k(src_hbm, dst_hbm, tn_hbm, zeros_hbm, out_hbm,
                  sidx, didx, rowsv, acc, gsem, ssem):
        c = lax.axis_index("c")
        s = lax.axis_index("s")
        wid = s * _NC + c
        start = wid * base_c + jnp.minimum(wid, extra)
        for j0 in range(0, base_c, 8):
            cnt = min(8, base_c - j0)
            batch = []
            for j in range(j0, j0 + cnt):
                batch.append(pltpu.async_copy(
                    src_hbm.at[pl.ds((start + j) * _CB, _CB)], sidx.at[j], gsem))
                batch.append(pltpu.async_copy(
                    dst_hbm.at[pl.ds((start + j) * _CB, _CB)], didx.at[j], gsem))
            for d in batch:
                d.wait()
        if extra:
            @pl.when(wid < extra)
            def _stage_extra():
                pltpu.async_copy(src_hbm.at[pl.ds((start + base_c) * _CB, _CB)],
                                 sidx.at[base_c], gsem).wait()
                pltpu.async_copy(dst_hbm.at[pl.ds((start + base_c) * _CB, _CB)],
                                 didx.at[base_c], gsem).wait()
        pltpu.sync_copy(zeros_hbm.at[pl.ds(s * rows_per_s, rows_per_s)],
                        acc.at[pl.ds(s * rows_per_s, rows_per_s)])
        plsc.subcore_barrier()

        def run_group(j0, cnt):
            loads = [pltpu.async_copy(tn_hbm.at[sidx.at[j0 + r]],
                                      rowsv.at[r], gsem)
                     for r in range(cnt)]
            stores = []
            for r in range(cnt):
                loads[r].wait()
                stores.append(pltpu.async_copy(
                    rowsv.at[r], acc.at[didx.at[j0 + r]], ssem, add=True))
            for d in stores:
                d.wait()

        def body(g, carry):
            run_group(g * _GRP, _GRP)
            return carry

        lax.fori_loop(0, groups, body, 0)
        if tail:
            run_group(groups * _GRP, tail)
        if extra:
            @pl.when(wid < extra)
            def _extra_chunk():
                run_group(base_c, 1)

        plsc.subcore_barrier()
        pltpu.sync_copy(acc.at[pl.ds(s * rows_per_s, rows_per_s)],
                        out_hbm.at[c, pl.ds(s * rows_per_s, rows_per_s)])

    return scatter_k


def _make_te_scatter_kernel(n_pad, total_chunks):
    base_c, extra, max_c, groups, tail = _chunk_split(total_chunks)
    rows_per_s = n_pad // _NS

    @functools.partial(
        pl.kernel,
        out_type=jax.ShapeDtypeStruct((_NC, n_pad, 32), jnp.float32),
        mesh=_make_mesh(),
        scratch_types=[
            pltpu.VMEM((max_c, _CB), jnp.int32),
            pltpu.VMEM((_GRP, _CB, 32), jnp.float32),
            pltpu.VMEM_SHARED((n_pad, 32), jnp.float32),
            pltpu.SemaphoreType.DMA,
            pltpu.SemaphoreType.DMA,
        ],
        compiler_params=pltpu.CompilerParams(use_tc_tiling_on_sc=False,
                                             disable_bounds_checks=True),
    )
    def scatter_k(dst_hbm, te_hbm, zeros_hbm, out_hbm,
                  didx, rowsv, acc, gsem, ssem):
        c = lax.axis_index("c")
        s = lax.axis_index("s")
        wid = s * _NC + c
        start = wid * base_c + jnp.minimum(wid, extra)
        for j0 in range(0, base_c, 16):
            cnt = min(16, base_c - j0)
            batch = [pltpu.async_copy(
                dst_hbm.at[pl.ds((start + j) * _CB, _CB)], didx.at[j], gsem)
                for j in range(j0, j0 + cnt)]
            for d in batch:
                d.wait()
        if extra:
            @pl.when(wid < extra)
            def _stage_extra():
                pltpu.async_copy(dst_hbm.at[pl.ds((start + base_c) * _CB, _CB)],
                                 didx.at[base_c], gsem).wait()
        pltpu.sync_copy(zeros_hbm.at[pl.ds(s * rows_per_s, rows_per_s)],
                        acc.at[pl.ds(s * rows_per_s, rows_per_s)])
        plsc.subcore_barrier()

        def run_group(j0, cnt):
            loads = [pltpu.async_copy(
                te_hbm.at[pl.ds((start + j0 + r) * _CB, _CB)],
                rowsv.at[r], gsem) for r in range(cnt)]
            stores = []
            for r in range(cnt):
                loads[r].wait()
                stores.append(pltpu.async_copy(
                    rowsv.at[r], acc.at[didx.at[j0 + r]], ssem, add=True))
            for d in stores:
                d.wait()

        def body(g, carry):
            run_group(g * _GRP, _GRP)
            return carry

        lax.fori_loop(0, groups, body, 0)
        if tail:
            run_group(groups * _GRP, tail)
        if extra:
            @pl.when(wid < extra)
            def _extra_chunk():
                run_group(base_c, 1)

        plsc.subcore_barrier()
        pltpu.sync_copy(acc.at[pl.ds(s * rows_per_s, rows_per_s)],
                        out_hbm.at[c, pl.ds(s * rows_per_s, rows_per_s)])

    return scatter_k


def _make_gather_kernel(n_nodes, p_pad):
    per_w = p_pad // _NW
    groups = per_w // 16
    mesh = plsc.VectorSubcoreMesh(core_axis_name="c", subcore_axis_name="s",
                                  num_cores=_NC, num_subcores=_NS)

    @functools.partial(
        pl.kernel,
        out_type=jax.ShapeDtypeStruct((p_pad,), jnp.float32),
        mesh=mesh,
        scratch_types=[
            pltpu.VMEM((n_nodes,), jnp.float32),
            pltpu.VMEM((per_w,), jnp.int32),
            pltpu.VMEM((per_w,), jnp.float32),
        ],
        compiler_params=pltpu.CompilerParams(needs_layout_passes=False,
                                             disable_bounds_checks=True),
    )
    def gather_k(z_hbm, pm_hbm, out_hbm, zv, idxv, outv):
        c = lax.axis_index("c")
        s = lax.axis_index("s")
        wid = s * _NC + c
        pltpu.sync_copy(z_hbm, zv)
        pltpu.sync_copy(pm_hbm.at[pl.ds(wid * per_w, per_w)], idxv)
        for g in range(groups):
            idx = idxv[pl.ds(g * 16, 16)]
            outv[pl.ds(g * 16, 16)] = plsc.load_gather(zv, [idx])
        pltpu.sync_copy(outv, out_hbm.at[pl.ds(wid * per_w, per_w)])

    return gather_k


def kernel(node_features, edge_index, edge_attr, post_mask,
           W_node, b_node, W_edge, b_edge, W_conv, b_conv, W_out, b_out):
    n, d_node = node_features.shape
    e = edge_attr.shape[0]
    d_edge = edge_attr.shape[1]
    h = W_node.shape[0]
    p = post_mask.shape[0]

    total_chunks = e // _CB
    n_pad = -(-(n + 1) // (8 * _NS)) * (8 * _NS)
    p_pad = -(-p // (16 * _NW)) * (16 * _NW)

    wnt = W_node.T
    wet = W_edge.T
    a_mat = W_conv[:, :h].T
    b_mat = W_conv[:, h:].T
    bn2 = b_node.reshape(1, h)
    be2 = b_edge.reshape(1, h)
    bc2 = b_conv.reshape(1, h)
    w2 = W_out.reshape(1, h)
    bo2 = b_out.reshape(1, 1)

    src = edge_index[0]
    dst = edge_index[1]
    ep8 = edge_attr.reshape(e // 8, 8 * d_edge)
    bd1 = jnp.kron(jnp.eye(8, dtype=jnp.float32), wet)
    bd2 = jnp.kron(jnp.eye(8, dtype=jnp.float32), b_mat)
    be8 = jnp.tile(b_edge, 8).reshape(1, 8 * h)
    bc8 = jnp.tile(b_conv, 8).reshape(1, 8 * h)
    pm = jnp.pad(post_mask, (0, p_pad - p))
    zeros_acc = jnp.zeros((n_pad, 32), jnp.float32)

    nb = 2048
    ng = -(-n // nb)
    node_emb, tn = pl.pallas_call(
        _node_body,
        grid=(ng,),
        in_specs=[
            pl.BlockSpec((nb, d_node), lambda i: (i, 0)),
            pl.BlockSpec((d_node, h), lambda i: (0, 0)),
            pl.BlockSpec((1, h), lambda i: (0, 0)),
            pl.BlockSpec((h, h), lambda i: (0, 0)),
        ],
        out_specs=[pl.BlockSpec((nb, h), lambda i: (i, 0)),
                   pl.BlockSpec((nb, h), lambda i: (i, 0))],
        out_shape=[jax.ShapeDtypeStruct((n, h), jnp.float32),
                   jax.ShapeDtypeStruct((n, h), jnp.float32)],
    )(node_features, wnt, bn2, a_mat)

    partials_n = _make_tn_scatter_kernel(n_pad, total_chunks)(
        src, dst, tn, zeros_acc)

    eb = 800
    eg = (e // 8) // eb
    te = pl.pallas_call(
        _edge_body,
        grid=(eg,),
        in_specs=[
            pl.BlockSpec((eb, 8 * d_edge), lambda i: (i, 0)),
            pl.BlockSpec((8 * d_edge, 8 * h), lambda i: (0, 0)),
            pl.BlockSpec((1, 8 * h), lambda i: (0, 0)),
            pl.BlockSpec((8 * h, 8 * h), lambda i: (0, 0)),
            pl.BlockSpec((1, 8 * h), lambda i: (0, 0)),
        ],
        out_specs=pl.BlockSpec((2 * eb, 4 * h), lambda i: (i, 0)),
        out_shape=jax.ShapeDtypeStruct((e // 4, 4 * h), jnp.float32),
    )(ep8, bd1, be8, bd2, bc8)

    partials_e = _make_te_scatter_kernel(n_pad, total_chunks)(
        dst, te.reshape(e, h), zeros_acc)

    fb = 1024
    fg = -(-n_pad // fb)
    z = pl.pallas_call(
        _final_body,
        grid=(fg,),
        in_specs=[
            pl.BlockSpec((fb, h), lambda i: (i, 0)),
            pl.BlockSpec((fb, h), lambda i: (i, 0)),
            pl.BlockSpec((fb, h), lambda i: (i, 0)),
            pl.BlockSpec((fb, h), lambda i: (i, 0)),
            pl.BlockSpec((fb, h), lambda i: (i, 0)),
            pl.BlockSpec((1, h), lambda i: (0, 0)),
            pl.BlockSpec((1, 1), lambda i: (0, 0)),
        ],
        out_specs=pl.BlockSpec((fb, 1), lambda i: (i, 0)),
        out_shape=jax.ShapeDtypeStruct((n, 1), jnp.float32),
    )(node_emb, partials_n[0], partials_n[1], partials_e[0], partials_e[1],
      w2, bo2)

    out = _make_gather_kernel(n, p_pad)(z.reshape(n), pm)
    return out[:p]


# speedup vs baseline: 1.0364x; 1.0364x over previous
import functools

import jax
import jax.numpy as jnp
from jax import lax
from jax.experimental import pallas as pl
from jax.experimental.pallas import tpu as pltpu
from jax.experimental.pallas import tpu_sc as plsc


def _node_body(x_ref, wnt_ref, bn_ref, a_ref, ne_ref, tn_ref):
    h = jnp.dot(x_ref[...], wnt_ref[...], preferred_element_type=jnp.float32)
    h = jnp.maximum(h + bn_ref[...], 0.0)
    ne_ref[...] = h
    tn_ref[...] = jnp.dot(h, a_ref[...], preferred_element_type=jnp.float32)


def _edge_body(e_ref, wet_ref, be_ref, bmat_ref, bc_ref, te_ref):
    h = jnp.dot(e_ref[...], wet_ref[...], preferred_element_type=jnp.float32)
    h = jnp.maximum(h + be_ref[...], 0.0)
    t = jnp.dot(h, bmat_ref[...], preferred_element_type=jnp.float32) + bc_ref[...]
    te_ref[...] = t.reshape(te_ref.shape)


def _final_body(ne_ref, p0_ref, p1_ref, w_ref, b_ref, z_ref):
    h = ne_ref[...] + p0_ref[...] + p1_ref[...]
    z = jnp.sum(h * w_ref[...], axis=1, keepdims=True) + b_ref[...]
    z_ref[...] = jax.nn.sigmoid(z)


_NC = 2
_NS = 16
_NW = _NC * _NS
_CB = 128
_GRP = 8


def _chunk_split(total_chunks):
    base_c = total_chunks // _NW
    extra = total_chunks - base_c * _NW
    max_c = base_c + (1 if extra else 0)
    groups = base_c // _GRP
    tail = base_c - groups * _GRP
    return base_c, extra, max_c, groups, tail


def _make_mesh():
    return plsc.VectorSubcoreMesh(core_axis_name="c", subcore_axis_name="s",
                                  num_cores=_NC, num_subcores=_NS)


def _make_tn_scatter_kernel(n_pad, total_chunks):
    base_c, extra, max_c, groups, tail = _chunk_split(total_chunks)
    rows_per_s = n_pad // _NS

    @functools.partial(
        pl.kernel,
        out_type=jax.ShapeDtypeStruct((_NC, n_pad, 32), jnp.float32),
        mesh=_make_mesh(),
        scratch_types=[
            pltpu.VMEM((max_c, _CB), jnp.int32),
            pltpu.VMEM((max_c, _CB), jnp.int32),
            pltpu.VMEM((_GRP, _CB, 32), jnp.float32),
            pltpu.VMEM_SHARED((n_pad, 32), jnp.float32),
            pltpu.SemaphoreType.DMA,
            pltpu.SemaphoreType.DMA,
        ],
        compiler_params=pltpu.CompilerParams(use_tc_tiling_on_sc=False,
                                             disable_bounds_checks=True),
    )
    def scatter_k(src_hbm, dst_hbm, tn_hbm, zeros_hbm, out_hbm,
                  sidx, didx, rowsv, acc, gsem, ssem):
        c = lax.axis_index("c")
        s = lax.axis_index("s")
        wid = s * _NC + c
        start = wid * base_c + jnp.minimum(wid, extra)
        pltpu.sync_copy(zeros_hbm.at[pl.ds(s * rows_per_s, rows_per_s)],
                        acc.at[pl.ds(s * rows_per_s, rows_per_s)])

        @pl.when(wid < extra)
        def _stage_big():
            pltpu.sync_copy(src_hbm.at[pl.ds(start, max_c)], sidx)
            pltpu.sync_copy(dst_hbm.at[pl.ds(start, max_c)], didx)

        @pl.when(wid >= extra)
        def _stage_small():
            pltpu.sync_copy(src_hbm.at[pl.ds(start, base_c)],
                            sidx.at[pl.ds(0, base_c)])
            pltpu.sync_copy(dst_hbm.at[pl.ds(start, base_c)],
                            didx.at[pl.ds(0, base_c)])

        plsc.subcore_barrier()

        def run_group(j0, cnt):
            loads = [pltpu.async_copy(tn_hbm.at[sidx.at[j0 + r]],
                                      rowsv.at[r], gsem)
                     for r in range(cnt)]
            stores = []
            for r in range(cnt):
                loads[r].wait()
                stores.append(pltpu.async_copy(
                    rowsv.at[r], acc.at[didx.at[j0 + r]], ssem, add=True))
            for d in stores:
                d.wait()

        def body(g, carry):
            run_group(g * _GRP, _GRP)
            return carry

        lax.fori_loop(0, groups, body, 0)
        if tail:
            run_group(groups * _GRP, tail)
        if extra:
            @pl.when(wid < extra)
            def _extra_chunk():
                run_group(base_c, 1)

        plsc.subcore_barrier()
        pltpu.sync_copy(acc.at[pl.ds(s * rows_per_s, rows_per_s)],
                        out_hbm.at[c, pl.ds(s * rows_per_s, rows_per_s)])

    return scatter_k


def _make_te_scatter_kernel(n_pad, total_chunks):
    base_c, extra, max_c, groups, tail = _chunk_split(total_chunks)
    rows_per_s = n_pad // _NS

    @functools.partial(
        pl.kernel,
        out_type=jax.ShapeDtypeStruct((_NC, n_pad, 32), jnp.float32),
        mesh=_make_mesh(),
        scratch_types=[
            pltpu.VMEM((max_c, _CB), jnp.int32),
            pltpu.VMEM((_GRP, _CB, 32), jnp.float32),
            pltpu.VMEM_SHARED((n_pad, 32), jnp.float32),
            pltpu.SemaphoreType.DMA,
            pltpu.SemaphoreType.DMA,
        ],
        compiler_params=pltpu.CompilerParams(use_tc_tiling_on_sc=False,
                                             disable_bounds_checks=True),
    )
    def scatter_k(dst_hbm, te_hbm, init_hbm, out_hbm,
                  didx, rowsv, acc, gsem, ssem):
        c = lax.axis_index("c")
        s = lax.axis_index("s")
        wid = s * _NC + c
        start = wid * base_c + jnp.minimum(wid, extra)
        pltpu.sync_copy(init_hbm.at[c, pl.ds(s * rows_per_s, rows_per_s)],
                        acc.at[pl.ds(s * rows_per_s, rows_per_s)])

        @pl.when(wid < extra)
        def _stage_big():
            pltpu.sync_copy(dst_hbm.at[pl.ds(start, max_c)], didx)

        @pl.when(wid >= extra)
        def _stage_small():
            pltpu.sync_copy(dst_hbm.at[pl.ds(start, base_c)],
                            didx.at[pl.ds(0, base_c)])

        plsc.subcore_barrier()

        def run_group(j0, cnt):
            loads = [pltpu.async_copy(
                te_hbm.at[pl.ds((start + j0 + r) * _CB, _CB)],
                rowsv.at[r], gsem) for r in range(cnt)]
            stores = []
            for r in range(cnt):
                loads[r].wait()
                stores.append(pltpu.async_copy(
                    rowsv.at[r], acc.at[didx.at[j0 + r]], ssem, add=True))
            for d in stores:
                d.wait()

        def body(g, carry):
            run_group(g * _GRP, _GRP)
            return carry

        lax.fori_loop(0, groups, body, 0)
        if tail:
            run_group(groups * _GRP, tail)
        if extra:
            @pl.when(wid < extra)
            def _extra_chunk():
                run_group(base_c, 1)

        plsc.subcore_barrier()
        pltpu.sync_copy(acc.at[pl.ds(s * rows_per_s, rows_per_s)],
                        out_hbm.at[c, pl.ds(s * rows_per_s, rows_per_s)])

    return scatter_k


def _make_gather_kernel(n_nodes, p_pad):
    per_w = p_pad // _NW
    groups = per_w // 16
    mesh = plsc.VectorSubcoreMesh(core_axis_name="c", subcore_axis_name="s",
                                  num_cores=_NC, num_subcores=_NS)

    @functools.partial(
        pl.kernel,
        out_type=jax.ShapeDtypeStruct((p_pad,), jnp.float32),
        mesh=mesh,
        scratch_types=[
            pltpu.VMEM((n_nodes,), jnp.float32),
            pltpu.VMEM((per_w,), jnp.int32),
            pltpu.VMEM((per_w,), jnp.float32),
        ],
        compiler_params=pltpu.CompilerParams(needs_layout_passes=False,
                                             disable_bounds_checks=True),
    )
    def gather_k(z_hbm, pm_hbm, out_hbm, zv, idxv, outv):
        c = lax.axis_index("c")
        s = lax.axis_index("s")
        wid = s * _NC + c
        pltpu.sync_copy(z_hbm, zv)
        pltpu.sync_copy(pm_hbm.at[pl.ds(wid * per_w, per_w)], idxv)
        for g in range(groups):
            idx = idxv[pl.ds(g * 16, 16)]
            outv[pl.ds(g * 16, 16)] = plsc.load_gather(zv, [idx])
        pltpu.sync_copy(outv, out_hbm.at[pl.ds(wid * per_w, per_w)])

    return gather_k


def kernel(node_features, edge_index, edge_attr, post_mask,
           W_node, b_node, W_edge, b_edge, W_conv, b_conv, W_out, b_out):
    n, d_node = node_features.shape
    e = edge_attr.shape[0]
    d_edge = edge_attr.shape[1]
    h = W_node.shape[0]
    p = post_mask.shape[0]

    total_chunks = e // _CB
    n_pad = -(-(n + 1) // (8 * _NS)) * (8 * _NS)
    p_pad = -(-p // (16 * _NW)) * (16 * _NW)

    wnt = W_node.T
    wet = W_edge.T
    a_mat = W_conv[:, :h].T
    b_mat = W_conv[:, h:].T
    bn2 = b_node.reshape(1, h)
    be2 = b_edge.reshape(1, h)
    bc2 = b_conv.reshape(1, h)
    w2 = W_out.reshape(1, h)
    bo2 = b_out.reshape(1, 1)

    src = edge_index[0]
    dst = edge_index[1]
    ep8 = edge_attr.reshape(e // 8, 8 * d_edge)
    bd1 = jnp.kron(jnp.eye(8, dtype=jnp.float32), wet)
    bd2 = jnp.kron(jnp.eye(8, dtype=jnp.float32), b_mat)
    be8 = jnp.tile(b_edge, 8).reshape(1, 8 * h)
    bc8 = jnp.tile(b_conv, 8).reshape(1, 8 * h)
    pm = jnp.pad(post_mask, (0, p_pad - p))
    zeros_acc = jnp.zeros((n_pad, 32), jnp.float32)

    nb = 2048
    ng = -(-n // nb)
    node_emb, tn = pl.pallas_call(
        _node_body,
        grid=(ng,),
        in_specs=[
            pl.BlockSpec((nb, d_node), lambda i: (i, 0)),
            pl.BlockSpec((d_node, h), lambda i: (0, 0)),
            pl.BlockSpec((1, h), lambda i: (0, 0)),
            pl.BlockSpec((h, h), lambda i: (0, 0)),
        ],
        out_specs=[pl.BlockSpec((nb, h), lambda i: (i, 0)),
                   pl.BlockSpec((nb, h), lambda i: (i, 0))],
        out_shape=[jax.ShapeDtypeStruct((n, h), jnp.float32),
                   jax.ShapeDtypeStruct((n, h), jnp.float32)],
    )(node_features, wnt, bn2, a_mat)

    src2 = src.reshape(total_chunks, _CB)
    dst2 = dst.reshape(total_chunks, _CB)
    partials_n = _make_tn_scatter_kernel(n_pad, total_chunks)(
        src2, dst2, tn, zeros_acc)

    eb = 800
    eg = (e // 8) // eb
    te = pl.pallas_call(
        _edge_body,
        grid=(eg,),
        in_specs=[
            pl.BlockSpec((eb, 8 * d_edge), lambda i: (i, 0)),
            pl.BlockSpec((8 * d_edge, 8 * h), lambda i: (0, 0)),
            pl.BlockSpec((1, 8 * h), lambda i: (0, 0)),
            pl.BlockSpec((8 * h, 8 * h), lambda i: (0, 0)),
            pl.BlockSpec((1, 8 * h), lambda i: (0, 0)),
        ],
        out_specs=pl.BlockSpec((2 * eb, 4 * h), lambda i: (i, 0)),
        out_shape=jax.ShapeDtypeStruct((e // 4, 4 * h), jnp.float32),
    )(ep8, bd1, be8, bd2, bc8)

    partials = _make_te_scatter_kernel(n_pad, total_chunks)(
        dst2, te.reshape(e, h), partials_n)

    fb = 1024
    fg = -(-n_pad // fb)
    z = pl.pallas_call(
        _final_body,
        grid=(fg,),
        in_specs=[
            pl.BlockSpec((fb, h), lambda i: (i, 0)),
            pl.BlockSpec((fb, h), lambda i: (i, 0)),
            pl.BlockSpec((fb, h), lambda i: (i, 0)),
            pl.BlockSpec((1, h), lambda i: (0, 0)),
            pl.BlockSpec((1, 1), lambda i: (0, 0)),
        ],
        out_specs=pl.BlockSpec((fb, 1), lambda i: (i, 0)),
        out_shape=jax.ShapeDtypeStruct((n, 1), jnp.float32),
    )(node_emb, partials[0], partials[1], w2, bo2)

    out = _make_gather_kernel(n, p_pad)(z.reshape(n), pm)
    return out[:p]


# speedup vs baseline: 1.0922x; 1.0538x over previous
import functools

import jax
import jax.numpy as jnp
from jax import lax
from jax.experimental import pallas as pl
from jax.experimental.pallas import tpu as pltpu
from jax.experimental.pallas import tpu_sc as plsc


def _node_body(x_ref, wnt_ref, bn_ref, a_ref, ne_ref, tn_ref):
    h = jnp.dot(x_ref[...], wnt_ref[...], preferred_element_type=jnp.float32)
    h = jnp.maximum(h + bn_ref[...], 0.0)
    ne_ref[...] = h
    tn_ref[...] = jnp.dot(h, a_ref[...], preferred_element_type=jnp.float32)


def _edge_body(e_ref, wet_ref, be_ref, bmat_ref, bc_ref, te_ref):
    h = jnp.dot(e_ref[...], wet_ref[...], preferred_element_type=jnp.float32)
    h = jnp.maximum(h + be_ref[...], 0.0)
    t = jnp.dot(h, bmat_ref[...], preferred_element_type=jnp.float32) + bc_ref[...]
    te_ref[...] = t.reshape(te_ref.shape)


def _final_body(ne_ref, p0_ref, p1_ref, w_ref, b_ref, z_ref):
    h = ne_ref[...] + p0_ref[...] + p1_ref[...]
    z = jnp.sum(h * w_ref[...], axis=1, keepdims=True) + b_ref[...]
    z_ref[...] = jax.nn.sigmoid(z)


_NC = 2
_NS = 16
_NW = _NC * _NS
_CB = 128
_GRP = 8


def _chunk_split(total_chunks):
    base_c = total_chunks // _NW
    extra = total_chunks - base_c * _NW
    max_c = base_c + (1 if extra else 0)
    groups = base_c // _GRP
    tail = base_c - groups * _GRP
    return base_c, extra, max_c, groups, tail


def _make_mesh():
    return plsc.VectorSubcoreMesh(core_axis_name="c", subcore_axis_name="s",
                                  num_cores=_NC, num_subcores=_NS)


def _make_tn_scatter_kernel(n_pad, total_chunks):
    base_c, extra, max_c, groups, tail = _chunk_split(total_chunks)
    rows_per_s = n_pad // _NS

    @functools.partial(
        pl.kernel,
        out_type=jax.ShapeDtypeStruct((_NC, n_pad, 32), jnp.float32),
        mesh=_make_mesh(),
        scratch_types=[
            pltpu.VMEM((max_c, _CB), jnp.int32),
            pltpu.VMEM((max_c, _CB), jnp.int32),
            pltpu.VMEM((_GRP, _CB, 32), jnp.float32),
            pltpu.VMEM_SHARED((n_pad, 32), jnp.float32),
            pltpu.SemaphoreType.DMA,
            pltpu.SemaphoreType.DMA,
        ],
        compiler_params=pltpu.CompilerParams(use_tc_tiling_on_sc=False,
                                             disable_bounds_checks=True),
    )
    def scatter_k(src_hbm, dst_hbm, tn_hbm, zeros_hbm, out_hbm,
                  sidx, didx, rowsv, acc, gsem, ssem):
        c = lax.axis_index("c")
        s = lax.axis_index("s")
        wid = s * _NC + c
        start = wid * base_c + jnp.minimum(wid, extra)
        pltpu.sync_copy(zeros_hbm.at[pl.ds(s * rows_per_s, rows_per_s)],
                        acc.at[pl.ds(s * rows_per_s, rows_per_s)])

        @pl.when(wid < extra)
        def _stage_big():
            pltpu.sync_copy(src_hbm.at[pl.ds(start, max_c)], sidx)
            pltpu.sync_copy(dst_hbm.at[pl.ds(start, max_c)], didx)

        @pl.when(wid >= extra)
        def _stage_small():
            pltpu.sync_copy(src_hbm.at[pl.ds(start, base_c)],
                            sidx.at[pl.ds(0, base_c)])
            pltpu.sync_copy(dst_hbm.at[pl.ds(start, base_c)],
                            didx.at[pl.ds(0, base_c)])

        plsc.subcore_barrier()

        def run_group(j0, cnt):
            loads = [pltpu.async_copy(tn_hbm.at[sidx.at[j0 + r]],
                                      rowsv.at[r], gsem)
                     for r in range(cnt)]
            stores = []
            for r in range(cnt):
                loads[r].wait()
                stores.append(pltpu.async_copy(
                    rowsv.at[r], acc.at[didx.at[j0 + r]], ssem, add=True))
            for d in stores:
                d.wait()

        def body(g, carry):
            run_group(g * _GRP, _GRP)
            return carry

        lax.fori_loop(0, groups, body, 0)
        if tail:
            run_group(groups * _GRP, tail)
        if extra:
            @pl.when(wid < extra)
            def _extra_chunk():
                run_group(base_c, 1)

        plsc.subcore_barrier()
        pltpu.sync_copy(acc.at[pl.ds(s * rows_per_s, rows_per_s)],
                        out_hbm.at[c, pl.ds(s * rows_per_s, rows_per_s)])

    return scatter_k


def _make_te_scatter_kernel(n_pad, total_chunks):
    base_c, extra, max_c, groups, tail = _chunk_split(total_chunks)
    rows_per_s = n_pad // _NS

    @functools.partial(
        pl.kernel,
        out_type=jax.ShapeDtypeStruct((_NC, n_pad, 32), jnp.float32),
        mesh=_make_mesh(),
        scratch_types=[
            pltpu.VMEM((max_c, _CB), jnp.int32),
            pltpu.VMEM((_GRP, _CB, 32), jnp.float32),
            pltpu.VMEM_SHARED((n_pad, 32), jnp.float32),
            pltpu.SemaphoreType.DMA,
            pltpu.SemaphoreType.DMA,
        ],
        compiler_params=pltpu.CompilerParams(use_tc_tiling_on_sc=False,
                                             disable_bounds_checks=True),
    )
    def scatter_k(dst_hbm, te_hbm, init_hbm, out_hbm,
                  didx, rowsv, acc, gsem, ssem):
        c = lax.axis_index("c")
        s = lax.axis_index("s")
        wid = s * _NC + c
        start = wid * base_c + jnp.minimum(wid, extra)
        pltpu.sync_copy(init_hbm.at[c, pl.ds(s * rows_per_s, rows_per_s)],
                        acc.at[pl.ds(s * rows_per_s, rows_per_s)])

        @pl.when(wid < extra)
        def _stage_big():
            pltpu.sync_copy(dst_hbm.at[pl.ds(start, max_c)], didx)

        @pl.when(wid >= extra)
        def _stage_small():
            pltpu.sync_copy(dst_hbm.at[pl.ds(start, base_c)],
                            didx.at[pl.ds(0, base_c)])

        plsc.subcore_barrier()

        def run_group(j0, cnt):
            loads = [pltpu.async_copy(
                te_hbm.at[pl.ds((start + j0 + r) * _CB, _CB)],
                rowsv.at[r], gsem) for r in range(cnt)]
            stores = []
            for r in range(cnt):
                loads[r].wait()
                stores.append(pltpu.async_copy(
                    rowsv.at[r], acc.at[didx.at[j0 + r]], ssem, add=True))
            for d in stores:
                d.wait()

        def body(g, carry):
            run_group(g * _GRP, _GRP)
            return carry

        lax.fori_loop(0, groups, body, 0)
        if tail:
            run_group(groups * _GRP, tail)
        if extra:
            @pl.when(wid < extra)
            def _extra_chunk():
                run_group(base_c, 1)

        plsc.subcore_barrier()
        pltpu.sync_copy(acc.at[pl.ds(s * rows_per_s, rows_per_s)],
                        out_hbm.at[c, pl.ds(s * rows_per_s, rows_per_s)])

    return scatter_k


def _make_gather_kernel(n_nodes, p_pad):
    per_w = p_pad // _NW
    groups = per_w // 16
    mesh = plsc.VectorSubcoreMesh(core_axis_name="c", subcore_axis_name="s",
                                  num_cores=_NC, num_subcores=_NS)

    @functools.partial(
        pl.kernel,
        out_type=jax.ShapeDtypeStruct((p_pad,), jnp.float32),
        mesh=mesh,
        scratch_types=[
            pltpu.VMEM((n_nodes,), jnp.float32),
            pltpu.VMEM((per_w,), jnp.int32),
            pltpu.VMEM((per_w,), jnp.float32),
        ],
        compiler_params=pltpu.CompilerParams(needs_layout_passes=False,
                                             disable_bounds_checks=True),
    )
    def gather_k(z_hbm, pm_hbm, out_hbm, zv, idxv, outv):
        c = lax.axis_index("c")
        s = lax.axis_index("s")
        wid = s * _NC + c
        pltpu.sync_copy(z_hbm, zv)
        pltpu.sync_copy(pm_hbm.at[pl.ds(wid * per_w, per_w)], idxv)
        for g in range(groups):
            idx = idxv[pl.ds(g * 16, 16)]
            outv[pl.ds(g * 16, 16)] = plsc.load_gather(zv, [idx])
        pltpu.sync_copy(outv, out_hbm.at[pl.ds(wid * per_w, per_w)])

    return gather_k


def kernel(node_features, edge_index, edge_attr, post_mask,
           W_node, b_node, W_edge, b_edge, W_conv, b_conv, W_out, b_out):
    n, d_node = node_features.shape
    e = edge_attr.shape[0]
    d_edge = edge_attr.shape[1]
    h = W_node.shape[0]
    p = post_mask.shape[0]

    total_chunks = e // _CB
    n_pad = -(-(n + 1) // (8 * _NS)) * (8 * _NS)
    p_pad = -(-p // (16 * _NW)) * (16 * _NW)

    wnt = W_node.T
    wet = W_edge.T
    a_mat = W_conv[:, :h].T
    b_mat = W_conv[:, h:].T
    bn2 = b_node.reshape(1, h)
    be2 = b_edge.reshape(1, h)
    bc2 = b_conv.reshape(1, h)
    w2 = W_out.reshape(1, h)
    bo2 = b_out.reshape(1, 1)

    src = edge_index[0]
    dst = edge_index[1]
    ep8 = edge_attr.reshape(e // 8, 8 * d_edge)
    bd1 = jnp.kron(jnp.eye(8, dtype=jnp.float32), wet)
    bd2 = jnp.kron(jnp.eye(8, dtype=jnp.float32), b_mat)
    be8 = jnp.tile(b_edge, 8).reshape(1, 8 * h)
    bc8 = jnp.tile(b_conv, 8).reshape(1, 8 * h)
    pm = jnp.pad(post_mask, (0, p_pad - p))
    zeros_acc = jnp.zeros((n_pad, 32), jnp.float32)

    nb = 2048
    ng = -(-n // nb)
    node_emb, tn = pl.pallas_call(
        _node_body,
        grid=(ng,),
        in_specs=[
            pl.BlockSpec((nb, d_node), lambda i: (i, 0)),
            pl.BlockSpec((d_node, h), lambda i: (0, 0)),
            pl.BlockSpec((1, h), lambda i: (0, 0)),
            pl.BlockSpec((h, h), lambda i: (0, 0)),
        ],
        out_specs=[pl.BlockSpec((nb, h), lambda i: (i, 0)),
                   pl.BlockSpec((nb, h), lambda i: (i, 0))],
        out_shape=[jax.ShapeDtypeStruct((n, h), jnp.float32),
                   jax.ShapeDtypeStruct((n, h), jnp.float32)],
    )(node_features, wnt, bn2, a_mat)

    src2 = src.reshape(total_chunks, _CB)
    dst2 = dst.reshape(total_chunks, _CB)
    partials_n = _make_tn_scatter_kernel(n_pad, total_chunks)(
        src2, dst2, tn, zeros_acc)

    eb = 1600
    eg = (e // 8) // eb
    te = pl.pallas_call(
        _edge_body,
        grid=(eg,),
        in_specs=[
            pl.BlockSpec((eb, 8 * d_edge), lambda i: (i, 0)),
            pl.BlockSpec((8 * d_edge, 8 * h), lambda i: (0, 0)),
            pl.BlockSpec((1, 8 * h), lambda i: (0, 0)),
            pl.BlockSpec((8 * h, 8 * h), lambda i: (0, 0)),
            pl.BlockSpec((1, 8 * h), lambda i: (0, 0)),
        ],
        out_specs=pl.BlockSpec((2 * eb, 4 * h), lambda i: (i, 0)),
        out_shape=jax.ShapeDtypeStruct((e // 4, 4 * h), jnp.float32),
    )(ep8, bd1, be8, bd2, bc8)

    partials = _make_te_scatter_kernel(n_pad, total_chunks)(
        dst2, te.reshape(e, h), partials_n)

    fb = 1024
    fg = -(-n_pad // fb)
    z = pl.pallas_call(
        _final_body,
        grid=(fg,),
        in_specs=[
            pl.BlockSpec((fb, h), lambda i: (i, 0)),
            pl.BlockSpec((fb, h), lambda i: (i, 0)),
            pl.BlockSpec((fb, h), lambda i: (i, 0)),
            pl.BlockSpec((1, h), lambda i: (0, 0)),
            pl.BlockSpec((1, 1), lambda i: (0, 0)),
        ],
        out_specs=pl.BlockSpec((fb, 1), lambda i: (i, 0)),
        out_shape=jax.ShapeDtypeStruct((n, 1), jnp.float32),
    )(node_emb, partials[0], partials[1], w2, bo2)

    out = _make_gather_kernel(n, p_pad)(z.reshape(n), pm)
    return out[:p]


# speedup vs baseline: 1.1298x; 1.0344x over previous
import functools

import jax
import jax.numpy as jnp
from jax import lax
from jax.experimental import pallas as pl
from jax.experimental.pallas import tpu as pltpu
from jax.experimental.pallas import tpu_sc as plsc


def _node_body(x_ref, wnt_ref, bn_ref, a_ref, ne_ref, tn_ref):
    h = jnp.dot(x_ref[...], wnt_ref[...], preferred_element_type=jnp.float32)
    h = jnp.maximum(h + bn_ref[...], 0.0)
    ne_ref[...] = h
    tn_ref[...] = jnp.dot(h, a_ref[...], preferred_element_type=jnp.float32)


def _edge_body(e_ref, wet_ref, be_ref, bmat_ref, bc_ref, te_ref):
    h = jnp.dot(e_ref[...], wet_ref[...], preferred_element_type=jnp.float32)
    h = jnp.maximum(h + be_ref[...], 0.0)
    t = jnp.dot(h, bmat_ref[...], preferred_element_type=jnp.float32) + bc_ref[...]
    te_ref[...] = t.reshape(te_ref.shape)


def _final_body(ne_ref, p0_ref, p1_ref, w_ref, b_ref, z_ref):
    h = ne_ref[...] + p0_ref[...] + p1_ref[...]
    z = jnp.sum(h * w_ref[...], axis=1, keepdims=True) + b_ref[...]
    z_ref[...] = jax.nn.sigmoid(z)


_NC = 2
_NS = 16
_NW = _NC * _NS
_CB = 128
_GRP = 8


def _chunk_split(total_chunks):
    base_c = total_chunks // _NW
    extra = total_chunks - base_c * _NW
    max_c = base_c + (1 if extra else 0)
    groups = base_c // _GRP
    tail = base_c - groups * _GRP
    return base_c, extra, max_c, groups, tail


def _make_mesh():
    return plsc.VectorSubcoreMesh(core_axis_name="c", subcore_axis_name="s",
                                  num_cores=_NC, num_subcores=_NS)


def _make_tn_scatter_kernel(n_pad, total_chunks):
    base_c, extra, max_c, groups, tail = _chunk_split(total_chunks)
    rows_per_s = n_pad // _NS

    @functools.partial(
        pl.kernel,
        out_type=jax.ShapeDtypeStruct((_NC, n_pad, 32), jnp.float32),
        mesh=_make_mesh(),
        scratch_types=[
            pltpu.VMEM((max_c, _CB), jnp.int32),
            pltpu.VMEM((max_c, _CB), jnp.int32),
            pltpu.VMEM((_GRP, _CB, 32), jnp.float32),
            pltpu.VMEM_SHARED((n_pad, 32), jnp.float32),
            pltpu.SemaphoreType.DMA,
            pltpu.SemaphoreType.DMA,
        ],
        compiler_params=pltpu.CompilerParams(use_tc_tiling_on_sc=False,
                                             disable_bounds_checks=True),
    )
    def scatter_k(src_hbm, dst_hbm, tn_hbm, zeros_hbm, out_hbm,
                  sidx, didx, rowsv, acc, gsem, ssem):
        c = lax.axis_index("c")
        s = lax.axis_index("s")
        wid = s * _NC + c
        start = wid * base_c + jnp.minimum(wid, extra)
        pltpu.sync_copy(zeros_hbm.at[pl.ds(s * rows_per_s, rows_per_s)],
                        acc.at[pl.ds(s * rows_per_s, rows_per_s)])

        @pl.when(wid < extra)
        def _stage_big():
            pltpu.sync_copy(src_hbm.at[pl.ds(start, max_c)], sidx)
            pltpu.sync_copy(dst_hbm.at[pl.ds(start, max_c)], didx)

        @pl.when(wid >= extra)
        def _stage_small():
            pltpu.sync_copy(src_hbm.at[pl.ds(start, base_c)],
                            sidx.at[pl.ds(0, base_c)])
            pltpu.sync_copy(dst_hbm.at[pl.ds(start, base_c)],
                            didx.at[pl.ds(0, base_c)])

        plsc.subcore_barrier()

        def run_group(j0, cnt):
            loads = [pltpu.async_copy(tn_hbm.at[sidx.at[j0 + r]],
                                      rowsv.at[r], gsem)
                     for r in range(cnt)]
            stores = []
            for r in range(cnt):
                loads[r].wait()
                stores.append(pltpu.async_copy(
                    rowsv.at[r], acc.at[didx.at[j0 + r]], ssem, add=True))
            for d in stores:
                d.wait()

        def body(g, carry):
            run_group(g * _GRP, _GRP)
            return carry

        lax.fori_loop(0, groups, body, 0)
        if tail:
            run_group(groups * _GRP, tail)
        if extra:
            @pl.when(wid < extra)
            def _extra_chunk():
                run_group(base_c, 1)

        plsc.subcore_barrier()
        pltpu.sync_copy(acc.at[pl.ds(s * rows_per_s, rows_per_s)],
                        out_hbm.at[c, pl.ds(s * rows_per_s, rows_per_s)])

    return scatter_k


def _make_te_scatter_kernel(n_pad, total_chunks):
    base_c, extra, max_c, groups, tail = _chunk_split(total_chunks)
    rows_per_s = n_pad // _NS

    @functools.partial(
        pl.kernel,
        out_type=jax.ShapeDtypeStruct((_NC, n_pad, 32), jnp.float32),
        mesh=_make_mesh(),
        scratch_types=[
            pltpu.VMEM((max_c, _CB), jnp.int32),
            pltpu.VMEM((_GRP, _CB, 32), jnp.float32),
            pltpu.VMEM_SHARED((n_pad, 32), jnp.float32),
            pltpu.SemaphoreType.DMA,
            pltpu.SemaphoreType.DMA,
        ],
        compiler_params=pltpu.CompilerParams(use_tc_tiling_on_sc=False,
                                             disable_bounds_checks=True),
    )
    def scatter_k(dst_hbm, te_hbm, init_hbm, out_hbm,
                  didx, rowsv, acc, gsem, ssem):
        c = lax.axis_index("c")
        s = lax.axis_index("s")
        wid = s * _NC + c
        start = wid * base_c + jnp.minimum(wid, extra)
        pltpu.sync_copy(init_hbm.at[c, pl.ds(s * rows_per_s, rows_per_s)],
                        acc.at[pl.ds(s * rows_per_s, rows_per_s)])

        @pl.when(wid < extra)
        def _stage_big():
            pltpu.sync_copy(dst_hbm.at[pl.ds(start, max_c)], didx)

        @pl.when(wid >= extra)
        def _stage_small():
            pltpu.sync_copy(dst_hbm.at[pl.ds(start, base_c)],
                            didx.at[pl.ds(0, base_c)])

        plsc.subcore_barrier()

        def run_group(j0, cnt):
            loads = [pltpu.async_copy(
                te_hbm.at[pl.ds((start + j0 + r) * _CB, _CB)],
                rowsv.at[r], gsem) for r in range(cnt)]
            stores = []
            for r in range(cnt):
                loads[r].wait()
                stores.append(pltpu.async_copy(
                    rowsv.at[r], acc.at[didx.at[j0 + r]], ssem, add=True))
            for d in stores:
                d.wait()

        def body(g, carry):
            run_group(g * _GRP, _GRP)
            return carry

        lax.fori_loop(0, groups, body, 0)
        if tail:
            run_group(groups * _GRP, tail)
        if extra:
            @pl.when(wid < extra)
            def _extra_chunk():
                run_group(base_c, 1)

        plsc.subcore_barrier()
        pltpu.sync_copy(acc.at[pl.ds(s * rows_per_s, rows_per_s)],
                        out_hbm.at[c, pl.ds(s * rows_per_s, rows_per_s)])

    return scatter_k


def _make_gather_kernel(n_nodes, p_pad):
    per_w = p_pad // _NW
    groups = per_w // 16
    mesh = plsc.VectorSubcoreMesh(core_axis_name="c", subcore_axis_name="s",
                                  num_cores=_NC, num_subcores=_NS)

    @functools.partial(
        pl.kernel,
        out_type=jax.ShapeDtypeStruct((p_pad,), jnp.float32),
        mesh=mesh,
        scratch_types=[
            pltpu.VMEM((n_nodes,), jnp.float32),
            pltpu.VMEM((per_w,), jnp.int32),
            pltpu.VMEM((per_w,), jnp.float32),
        ],
        compiler_params=pltpu.CompilerParams(needs_layout_passes=False,
                                             disable_bounds_checks=True),
    )
    def gather_k(z_hbm, pm_hbm, out_hbm, zv, idxv, outv):
        c = lax.axis_index("c")
        s = lax.axis_index("s")
        wid = s * _NC + c
        pltpu.sync_copy(z_hbm, zv)
        pltpu.sync_copy(pm_hbm.at[pl.ds(wid * per_w, per_w)], idxv)
        for g in range(groups):
            idx = idxv[pl.ds(g * 16, 16)]
            outv[pl.ds(g * 16, 16)] = plsc.load_gather(zv, [idx])
        pltpu.sync_copy(outv, out_hbm.at[pl.ds(wid * per_w, per_w)])

    return gather_k


def kernel(node_features, edge_index, edge_attr, post_mask,
           W_node, b_node, W_edge, b_edge, W_conv, b_conv, W_out, b_out):
    n, d_node = node_features.shape
    e = edge_attr.shape[0]
    d_edge = edge_attr.shape[1]
    h = W_node.shape[0]
    p = post_mask.shape[0]

    total_chunks = e // _CB
    n_pad = -(-(n + 1) // (8 * _NS)) * (8 * _NS)
    p_pad = -(-p // (16 * _NW)) * (16 * _NW)

    wnt = W_node.T
    wet = W_edge.T
    a_mat = W_conv[:, :h].T
    b_mat = W_conv[:, h:].T
    bn2 = b_node.reshape(1, h)
    be2 = b_edge.reshape(1, h)
    bc2 = b_conv.reshape(1, h)
    w2 = W_out.reshape(1, h)
    bo2 = b_out.reshape(1, 1)

    src = edge_index[0]
    dst = edge_index[1]
    ep8 = edge_attr.reshape(e // 8, 8 * d_edge)
    bd1 = jnp.kron(jnp.eye(8, dtype=jnp.float32), wet)
    bd2 = jnp.kron(jnp.eye(8, dtype=jnp.float32), b_mat)
    be8 = jnp.tile(b_edge, 8).reshape(1, 8 * h)
    bc8 = jnp.tile(b_conv, 8).reshape(1, 8 * h)
    pm = jnp.pad(post_mask, (0, p_pad - p))
    zeros_acc = jnp.zeros((n_pad, 32), jnp.float32)

    nb = 2048
    ng = -(-n // nb)
    node_emb, tn = pl.pallas_call(
        _node_body,
        grid=(ng,),
        in_specs=[
            pl.BlockSpec((nb, d_node), lambda i: (i, 0)),
            pl.BlockSpec((d_node, h), lambda i: (0, 0)),
            pl.BlockSpec((1, h), lambda i: (0, 0)),
            pl.BlockSpec((h, h), lambda i: (0, 0)),
        ],
        out_specs=[pl.BlockSpec((nb, h), lambda i: (i, 0)),
                   pl.BlockSpec((nb, h), lambda i: (i, 0))],
        out_shape=[jax.ShapeDtypeStruct((n, h), jnp.float32),
                   jax.ShapeDtypeStruct((n, h), jnp.float32)],
    )(node_features, wnt, bn2, a_mat)

    src2 = src.reshape(total_chunks, _CB)
    dst2 = dst.reshape(total_chunks, _CB)
    partials_n = _make_tn_scatter_kernel(n_pad, total_chunks)(
        src2, dst2, tn, zeros_acc)

    eb = 4000
    eg = (e // 8) // eb
    te = pl.pallas_call(
        _edge_body,
        grid=(eg,),
        in_specs=[
            pl.BlockSpec((eb, 8 * d_edge), lambda i: (i, 0)),
            pl.BlockSpec((8 * d_edge, 8 * h), lambda i: (0, 0)),
            pl.BlockSpec((1, 8 * h), lambda i: (0, 0)),
            pl.BlockSpec((8 * h, 8 * h), lambda i: (0, 0)),
            pl.BlockSpec((1, 8 * h), lambda i: (0, 0)),
        ],
        out_specs=pl.BlockSpec((2 * eb, 4 * h), lambda i: (i, 0)),
        out_shape=jax.ShapeDtypeStruct((e // 4, 4 * h), jnp.float32),
    )(ep8, bd1, be8, bd2, bc8)

    partials = _make_te_scatter_kernel(n_pad, total_chunks)(
        dst2, te.reshape(e, h), partials_n)

    fb = 1024
    fg = -(-n_pad // fb)
    z = pl.pallas_call(
        _final_body,
        grid=(fg,),
        in_specs=[
            pl.BlockSpec((fb, h), lambda i: (i, 0)),
            pl.BlockSpec((fb, h), lambda i: (i, 0)),
            pl.BlockSpec((fb, h), lambda i: (i, 0)),
            pl.BlockSpec((1, h), lambda i: (0, 0)),
            pl.BlockSpec((1, 1), lambda i: (0, 0)),
        ],
        out_specs=pl.BlockSpec((fb, 1), lambda i: (i, 0)),
        out_shape=jax.ShapeDtypeStruct((n, 1), jnp.float32),
    )(node_emb, partials[0], partials[1], w2, bo2)

    out = _make_gather_kernel(n, p_pad)(z.reshape(n), pm)
    return out[:p]


# speedup vs baseline: 1.1367x; 1.0061x over previous
import functools

import jax
import jax.numpy as jnp
from jax import lax
from jax.experimental import pallas as pl
from jax.experimental.pallas import tpu as pltpu
from jax.experimental.pallas import tpu_sc as plsc


def _node_body(x_ref, wnt_ref, bn_ref, a_ref, ne_ref, tn_ref):
    h = jnp.dot(x_ref[...], wnt_ref[...], preferred_element_type=jnp.float32)
    h = jnp.maximum(h + bn_ref[...], 0.0)
    ne_ref[...] = h
    tn_ref[...] = jnp.dot(h, a_ref[...], preferred_element_type=jnp.float32)


def _edge_body(e_ref, wet_ref, be_ref, bmat_ref, bc_ref, te_ref):
    h = jnp.dot(e_ref[...], wet_ref[...], preferred_element_type=jnp.float32)
    h = jnp.maximum(h + be_ref[...], 0.0)
    t = jnp.dot(h, bmat_ref[...], preferred_element_type=jnp.float32) + bc_ref[...]
    te_ref[...] = t.reshape(te_ref.shape)


def _final_body(ne_ref, p0_ref, p1_ref, w_ref, b_ref, z_ref):
    h = ne_ref[...] + p0_ref[...] + p1_ref[...]
    z = jnp.sum(h * w_ref[...], axis=1, keepdims=True) + b_ref[...]
    z_ref[...] = jax.nn.sigmoid(z)


_NC = 2
_NS = 16
_NW = _NC * _NS
_CB = 128
_GRP = 8


def _chunk_split(total_chunks):
    base_c = total_chunks // _NW
    extra = total_chunks - base_c * _NW
    max_c = base_c + (1 if extra else 0)
    groups = base_c // _GRP
    tail = base_c - groups * _GRP
    return base_c, extra, max_c, groups, tail


def _make_mesh():
    return plsc.VectorSubcoreMesh(core_axis_name="c", subcore_axis_name="s",
                                  num_cores=_NC, num_subcores=_NS)


def _make_tn_scatter_kernel(n_pad, total_chunks):
    base_c, extra, max_c, groups, tail = _chunk_split(total_chunks)
    rows_per_s = n_pad // _NS

    @functools.partial(
        pl.kernel,
        out_type=jax.ShapeDtypeStruct((_NC, n_pad, 32), jnp.float32),
        mesh=_make_mesh(),
        scratch_types=[
            pltpu.VMEM((max_c, _CB), jnp.int32),
            pltpu.VMEM((max_c, _CB), jnp.int32),
            pltpu.VMEM((_GRP, _CB, 32), jnp.float32),
            pltpu.VMEM_SHARED((n_pad, 32), jnp.float32),
            pltpu.SemaphoreType.DMA,
            pltpu.SemaphoreType.DMA,
        ],
        compiler_params=pltpu.CompilerParams(use_tc_tiling_on_sc=False,
                                             disable_bounds_checks=True),
    )
    def scatter_k(src_hbm, dst_hbm, tn_hbm, zeros_hbm, out_hbm,
                  sidx, didx, rowsv, acc, gsem, ssem):
        c = lax.axis_index("c")
        s = lax.axis_index("s")
        wid = s * _NC + c
        start = wid * base_c + jnp.minimum(wid, extra)
        pltpu.sync_copy(zeros_hbm.at[pl.ds(s * rows_per_s, rows_per_s)],
                        acc.at[pl.ds(s * rows_per_s, rows_per_s)])

        @pl.when(wid < extra)
        def _stage_big():
            pltpu.sync_copy(src_hbm.at[pl.ds(start, max_c)], sidx)
            pltpu.sync_copy(dst_hbm.at[pl.ds(start, max_c)], didx)

        @pl.when(wid >= extra)
        def _stage_small():
            pltpu.sync_copy(src_hbm.at[pl.ds(start, base_c)],
                            sidx.at[pl.ds(0, base_c)])
            pltpu.sync_copy(dst_hbm.at[pl.ds(start, base_c)],
                            didx.at[pl.ds(0, base_c)])

        plsc.subcore_barrier()

        def run_group(j0, cnt):
            loads = [pltpu.async_copy(tn_hbm.at[sidx.at[j0 + r]],
                                      rowsv.at[r], gsem)
                     for r in range(cnt)]
            stores = []
            for r in range(cnt):
                loads[r].wait()
                stores.append(pltpu.async_copy(
                    rowsv.at[r], acc.at[didx.at[j0 + r]], ssem, add=True))
            for d in stores:
                d.wait()

        def body(g, carry):
            run_group(g * _GRP, _GRP)
            return carry

        lax.fori_loop(0, groups, body, 0)
        if tail:
            run_group(groups * _GRP, tail)
        if extra:
            @pl.when(wid < extra)
            def _extra_chunk():
                run_group(base_c, 1)

        plsc.subcore_barrier()
        pltpu.sync_copy(acc.at[pl.ds(s * rows_per_s, rows_per_s)],
                        out_hbm.at[c, pl.ds(s * rows_per_s, rows_per_s)])

    return scatter_k


def _make_te_scatter_kernel(n_pad, total_chunks):
    base_c, extra, max_c, groups, tail = _chunk_split(total_chunks)
    rows_per_s = n_pad // _NS

    @functools.partial(
        pl.kernel,
        out_type=jax.ShapeDtypeStruct((_NC, n_pad, 32), jnp.float32),
        mesh=_make_mesh(),
        scratch_types=[
            pltpu.VMEM((max_c, _CB), jnp.int32),
            pltpu.VMEM((_GRP, _CB, 32), jnp.float32),
            pltpu.VMEM_SHARED((n_pad, 32), jnp.float32),
            pltpu.SemaphoreType.DMA,
            pltpu.SemaphoreType.DMA,
        ],
        compiler_params=pltpu.CompilerParams(use_tc_tiling_on_sc=False,
                                             disable_bounds_checks=True),
    )
    def scatter_k(dst_hbm, te_hbm, init_hbm, out_hbm,
                  didx, rowsv, acc, gsem, ssem):
        c = lax.axis_index("c")
        s = lax.axis_index("s")
        wid = s * _NC + c
        start = wid * base_c + jnp.minimum(wid, extra)
        pltpu.sync_copy(init_hbm.at[c, pl.ds(s * rows_per_s, rows_per_s)],
                        acc.at[pl.ds(s * rows_per_s, rows_per_s)])

        @pl.when(wid < extra)
        def _stage_big():
            pltpu.sync_copy(dst_hbm.at[pl.ds(start, max_c)], didx)

        @pl.when(wid >= extra)
        def _stage_small():
            pltpu.sync_copy(dst_hbm.at[pl.ds(start, base_c)],
                            didx.at[pl.ds(0, base_c)])

        plsc.subcore_barrier()

        def run_group(j0, cnt):
            loads = [pltpu.async_copy(
                te_hbm.at[pl.ds((start + j0 + r) * _CB, _CB)],
                rowsv.at[r], gsem) for r in range(cnt)]
            stores = []
            for r in range(cnt):
                loads[r].wait()
                stores.append(pltpu.async_copy(
                    rowsv.at[r], acc.at[didx.at[j0 + r]], ssem, add=True))
            for d in stores:
                d.wait()

        def body(g, carry):
            run_group(g * _GRP, _GRP)
            return carry

        lax.fori_loop(0, groups, body, 0)
        if tail:
            run_group(groups * _GRP, tail)
        if extra:
            @pl.when(wid < extra)
            def _extra_chunk():
                run_group(base_c, 1)

        plsc.subcore_barrier()
        pltpu.sync_copy(acc.at[pl.ds(s * rows_per_s, rows_per_s)],
                        out_hbm.at[c, pl.ds(s * rows_per_s, rows_per_s)])

    return scatter_k


def _make_gather_kernel(n_nodes, p_pad):
    per_w = p_pad // _NW
    groups = per_w // 16
    mesh = plsc.VectorSubcoreMesh(core_axis_name="c", subcore_axis_name="s",
                                  num_cores=_NC, num_subcores=_NS)

    @functools.partial(
        pl.kernel,
        out_type=jax.ShapeDtypeStruct((p_pad,), jnp.float32),
        mesh=mesh,
        scratch_types=[
            pltpu.VMEM((n_nodes,), jnp.float32),
            pltpu.VMEM((per_w,), jnp.int32),
            pltpu.VMEM((per_w,), jnp.float32),
        ],
        compiler_params=pltpu.CompilerParams(needs_layout_passes=False,
                                             disable_bounds_checks=True),
    )
    def gather_k(z_hbm, pm_hbm, out_hbm, zv, idxv, outv):
        c = lax.axis_index("c")
        s = lax.axis_index("s")
        wid = s * _NC + c
        pltpu.sync_copy(z_hbm, zv)
        pltpu.sync_copy(pm_hbm.at[pl.ds(wid * per_w, per_w)], idxv)
        for g in range(groups):
            idx = idxv[pl.ds(g * 16, 16)]
            outv[pl.ds(g * 16, 16)] = plsc.load_gather(zv, [idx])
        pltpu.sync_copy(outv, out_hbm.at[pl.ds(wid * per_w, per_w)])

    return gather_k


def kernel(node_features, edge_index, edge_attr, post_mask,
           W_node, b_node, W_edge, b_edge, W_conv, b_conv, W_out, b_out):
    n, d_node = node_features.shape
    e = edge_attr.shape[0]
    d_edge = edge_attr.shape[1]
    h = W_node.shape[0]
    p = post_mask.shape[0]

    total_chunks = e // _CB
    n_pad = -(-(n + 1) // (8 * _NS)) * (8 * _NS)
    p_pad = -(-p // (16 * _NW)) * (16 * _NW)

    wnt = W_node.T
    wet = W_edge.T
    a_mat = W_conv[:, :h].T
    b_mat = W_conv[:, h:].T
    bn2 = b_node.reshape(1, h)
    be2 = b_edge.reshape(1, h)
    bc2 = b_conv.reshape(1, h)
    w2 = W_out.reshape(1, h)
    bo2 = b_out.reshape(1, 1)

    src = edge_index[0]
    dst = edge_index[1]
    ep8 = edge_attr.reshape(e // 8, 8 * d_edge)
    bd1 = jnp.kron(jnp.eye(8, dtype=jnp.float32), wet)
    bd2 = jnp.kron(jnp.eye(8, dtype=jnp.float32), b_mat)
    be8 = jnp.tile(b_edge, 8).reshape(1, 8 * h)
    bc8 = jnp.tile(b_conv, 8).reshape(1, 8 * h)
    pm = jnp.pad(post_mask, (0, p_pad - p))
    zeros_acc = jnp.zeros((n_pad, 32), jnp.float32)

    nb = 2048
    ng = -(-n // nb)
    node_emb, tn = pl.pallas_call(
        _node_body,
        grid=(ng,),
        in_specs=[
            pl.BlockSpec((nb, d_node), lambda i: (i, 0)),
            pl.BlockSpec((d_node, h), lambda i: (0, 0)),
            pl.BlockSpec((1, h), lambda i: (0, 0)),
            pl.BlockSpec((h, h), lambda i: (0, 0)),
        ],
        out_specs=[pl.BlockSpec((nb, h), lambda i: (i, 0)),
                   pl.BlockSpec((nb, h), lambda i: (i, 0))],
        out_shape=[jax.ShapeDtypeStruct((n, h), jnp.float32),
                   jax.ShapeDtypeStruct((n, h), jnp.float32)],
    )(node_features, wnt, bn2, a_mat)

    src2 = src.reshape(total_chunks, _CB)
    dst2 = dst.reshape(total_chunks, _CB)
    partials_n = _make_tn_scatter_kernel(n_pad, total_chunks)(
        src2, dst2, tn, zeros_acc)

    eb = 8000
    eg = (e // 8) // eb
    te = pl.pallas_call(
        _edge_body,
        grid=(eg,),
        in_specs=[
            pl.BlockSpec((eb, 8 * d_edge), lambda i: (i, 0)),
            pl.BlockSpec((8 * d_edge, 8 * h), lambda i: (0, 0)),
            pl.BlockSpec((1, 8 * h), lambda i: (0, 0)),
            pl.BlockSpec((8 * h, 8 * h), lambda i: (0, 0)),
            pl.BlockSpec((1, 8 * h), lambda i: (0, 0)),
        ],
        out_specs=pl.BlockSpec((2 * eb, 4 * h), lambda i: (i, 0)),
        out_shape=jax.ShapeDtypeStruct((e // 4, 4 * h), jnp.float32),
    )(ep8, bd1, be8, bd2, bc8)

    partials = _make_te_scatter_kernel(n_pad, total_chunks)(
        dst2, te.reshape(e, h), partials_n)

    fb = 1024
    fg = -(-n_pad // fb)
    z = pl.pallas_call(
        _final_body,
        grid=(fg,),
        in_specs=[
            pl.BlockSpec((fb, h), lambda i: (i, 0)),
            pl.BlockSpec((fb, h), lambda i: (i, 0)),
            pl.BlockSpec((fb, h), lambda i: (i, 0)),
            pl.BlockSpec((1, h), lambda i: (0, 0)),
            pl.BlockSpec((1, 1), lambda i: (0, 0)),
        ],
        out_specs=pl.BlockSpec((fb, 1), lambda i: (i, 0)),
        out_shape=jax.ShapeDtypeStruct((n, 1), jnp.float32),
    )(node_emb, partials[0], partials[1], w2, bo2)

    out = _make_gather_kernel(n, p_pad)(z.reshape(n), pm)
    return out[:p]


# speedup vs baseline: 1.1370x; 1.0003x over previous
import functools

import jax
import jax.numpy as jnp
from jax import lax
from jax.experimental import pallas as pl
from jax.experimental.pallas import tpu as pltpu
from jax.experimental.pallas import tpu_sc as plsc


def _node_body(x_ref, wnt_ref, bn_ref, a_ref, ne_ref, tn_ref):
    h = jnp.dot(x_ref[...], wnt_ref[...], preferred_element_type=jnp.float32)
    h = jnp.maximum(h + bn_ref[...], 0.0)
    ne_ref[...] = h
    tn_ref[...] = jnp.dot(h, a_ref[...], preferred_element_type=jnp.float32)


def _edge_body(e_ref, wet_ref, be_ref, bmat_ref, bc_ref, te_ref):
    h = jnp.dot(e_ref[...], wet_ref[...], preferred_element_type=jnp.float32)
    h = jnp.maximum(h + be_ref[...], 0.0)
    t = jnp.dot(h, bmat_ref[...], preferred_element_type=jnp.float32) + bc_ref[...]
    te_ref[...] = t.reshape(te_ref.shape)


def _final_body(ne_ref, p0_ref, p1_ref, w_ref, b_ref, z_ref):
    h = ne_ref[...] + p0_ref[...] + p1_ref[...]
    z = jnp.sum(h * w_ref[...], axis=1, keepdims=True) + b_ref[...]
    z_ref[...] = jax.nn.sigmoid(z)


_NC = 2
_NS = 16
_NW = _NC * _NS
_CB = 128
_GRP = 8


def _chunk_split(total_chunks):
    base_c = total_chunks // _NW
    extra = total_chunks - base_c * _NW
    max_c = base_c + (1 if extra else 0)
    groups = base_c // _GRP
    tail = base_c - groups * _GRP
    return base_c, extra, max_c, groups, tail


def _make_mesh():
    return plsc.VectorSubcoreMesh(core_axis_name="c", subcore_axis_name="s",
                                  num_cores=_NC, num_subcores=_NS)


def _make_tn_scatter_kernel(n_pad, total_chunks):
    base_c, extra, max_c, groups, tail = _chunk_split(total_chunks)
    rows_per_s = n_pad // _NS

    @functools.partial(
        pl.kernel,
        out_type=jax.ShapeDtypeStruct((_NC, n_pad, 32), jnp.float32),
        mesh=_make_mesh(),
        scratch_types=[
            pltpu.VMEM((max_c, _CB), jnp.int32),
            pltpu.VMEM((max_c, _CB), jnp.int32),
            pltpu.VMEM((_GRP, _CB, 32), jnp.float32),
            pltpu.VMEM_SHARED((n_pad, 32), jnp.float32),
            pltpu.SemaphoreType.DMA,
            pltpu.SemaphoreType.DMA,
        ],
        compiler_params=pltpu.CompilerParams(use_tc_tiling_on_sc=False,
                                             disable_bounds_checks=True),
    )
    def scatter_k(src_hbm, dst_hbm, tn_hbm, zeros_hbm, out_hbm,
                  sidx, didx, rowsv, acc, gsem, ssem):
        c = lax.axis_index("c")
        s = lax.axis_index("s")
        wid = s * _NC + c
        start = wid * base_c + jnp.minimum(wid, extra)
        pltpu.sync_copy(zeros_hbm.at[pl.ds(s * rows_per_s, rows_per_s)],
                        acc.at[pl.ds(s * rows_per_s, rows_per_s)])

        @pl.when(wid < extra)
        def _stage_big():
            pltpu.sync_copy(src_hbm.at[pl.ds(start, max_c)], sidx)
            pltpu.sync_copy(dst_hbm.at[pl.ds(start, max_c)], didx)

        @pl.when(wid >= extra)
        def _stage_small():
            pltpu.sync_copy(src_hbm.at[pl.ds(start, base_c)],
                            sidx.at[pl.ds(0, base_c)])
            pltpu.sync_copy(dst_hbm.at[pl.ds(start, base_c)],
                            didx.at[pl.ds(0, base_c)])

        plsc.subcore_barrier()

        def run_group(j0, cnt):
            loads = [pltpu.async_copy(tn_hbm.at[sidx.at[j0 + r]],
                                      rowsv.at[r], gsem)
                     for r in range(cnt)]
            stores = []
            for r in range(cnt):
                loads[r].wait()
                stores.append(pltpu.async_copy(
                    rowsv.at[r], acc.at[didx.at[j0 + r]], ssem, add=True))
            for d in stores:
                d.wait()

        def body(g, carry):
            run_group(g * _GRP, _GRP)
            return carry

        lax.fori_loop(0, groups, body, 0)
        if tail:
            run_group(groups * _GRP, tail)
        if extra:
            @pl.when(wid < extra)
            def _extra_chunk():
                run_group(base_c, 1)

        plsc.subcore_barrier()
        pltpu.sync_copy(acc.at[pl.ds(s * rows_per_s, rows_per_s)],
                        out_hbm.at[c, pl.ds(s * rows_per_s, rows_per_s)])

    return scatter_k


def _make_te_scatter_kernel(n_pad, total_chunks):
    base_c, extra, max_c, groups, tail = _chunk_split(total_chunks)
    rows_per_s = n_pad // _NS

    @functools.partial(
        pl.kernel,
        out_type=jax.ShapeDtypeStruct((_NC, n_pad, 32), jnp.float32),
        mesh=_make_mesh(),
        scratch_types=[
            pltpu.VMEM((max_c, _CB), jnp.int32),
            pltpu.VMEM((_GRP, _CB, 32), jnp.float32),
            pltpu.VMEM_SHARED((n_pad, 32), jnp.float32),
            pltpu.SemaphoreType.DMA,
            pltpu.SemaphoreType.DMA,
        ],
        compiler_params=pltpu.CompilerParams(use_tc_tiling_on_sc=False,
                                             disable_bounds_checks=True),
    )
    def scatter_k(dst_hbm, te_hbm, init_hbm, out_hbm,
                  didx, rowsv, acc, gsem, ssem):
        c = lax.axis_index("c")
        s = lax.axis_index("s")
        wid = s * _NC + c
        start = wid * base_c + jnp.minimum(wid, extra)
        pltpu.sync_copy(init_hbm.at[c, pl.ds(s * rows_per_s, rows_per_s)],
                        acc.at[pl.ds(s * rows_per_s, rows_per_s)])

        @pl.when(wid < extra)
        def _stage_big():
            pltpu.sync_copy(dst_hbm.at[pl.ds(start, max_c)], didx)

        @pl.when(wid >= extra)
        def _stage_small():
            pltpu.sync_copy(dst_hbm.at[pl.ds(start, base_c)],
                            didx.at[pl.ds(0, base_c)])

        plsc.subcore_barrier()

        def run_group(j0, cnt):
            loads = [pltpu.async_copy(
                te_hbm.at[pl.ds((start + j0 + r) * _CB, _CB)],
                rowsv.at[r], gsem) for r in range(cnt)]
            stores = []
            for r in range(cnt):
                loads[r].wait()
                stores.append(pltpu.async_copy(
                    rowsv.at[r], acc.at[didx.at[j0 + r]], ssem, add=True))
            for d in stores:
                d.wait()

        def body(g, carry):
            run_group(g * _GRP, _GRP)
            return carry

        lax.fori_loop(0, groups, body, 0)
        if tail:
            run_group(groups * _GRP, tail)
        if extra:
            @pl.when(wid < extra)
            def _extra_chunk():
                run_group(base_c, 1)

        plsc.subcore_barrier()
        pltpu.sync_copy(acc.at[pl.ds(s * rows_per_s, rows_per_s)],
                        out_hbm.at[c, pl.ds(s * rows_per_s, rows_per_s)])

    return scatter_k


def _make_gather_kernel(n_nodes, p_pad):
    per_w = p_pad // _NW
    groups = per_w // 16
    mesh = plsc.VectorSubcoreMesh(core_axis_name="c", subcore_axis_name="s",
                                  num_cores=_NC, num_subcores=_NS)

    @functools.partial(
        pl.kernel,
        out_type=jax.ShapeDtypeStruct((p_pad,), jnp.float32),
        mesh=mesh,
        scratch_types=[
            pltpu.VMEM((n_nodes,), jnp.float32),
            pltpu.VMEM((per_w,), jnp.int32),
            pltpu.VMEM((per_w,), jnp.float32),
        ],
        compiler_params=pltpu.CompilerParams(needs_layout_passes=False,
                                             disable_bounds_checks=True),
    )
    def gather_k(z_hbm, pm_hbm, out_hbm, zv, idxv, outv):
        c = lax.axis_index("c")
        s = lax.axis_index("s")
        wid = s * _NC + c
        pltpu.sync_copy(z_hbm, zv)
        pltpu.sync_copy(pm_hbm.at[pl.ds(wid * per_w, per_w)], idxv)
        for g in range(groups):
            idx = idxv[pl.ds(g * 16, 16)]
            outv[pl.ds(g * 16, 16)] = plsc.load_gather(zv, [idx])
        pltpu.sync_copy(outv, out_hbm.at[pl.ds(wid * per_w, per_w)])

    return gather_k


def kernel(node_features, edge_index, edge_attr, post_mask,
           W_node, b_node, W_edge, b_edge, W_conv, b_conv, W_out, b_out):
    n, d_node = node_features.shape
    e = edge_attr.shape[0]
    d_edge = edge_attr.shape[1]
    h = W_node.shape[0]
    p = post_mask.shape[0]

    total_chunks = e // _CB
    n_pad = -(-(n + 1) // (8 * _NS)) * (8 * _NS)
    p_pad = -(-p // (16 * _NW)) * (16 * _NW)

    wnt = W_node.T
    wet = W_edge.T
    a_mat = W_conv[:, :h].T
    b_mat = W_conv[:, h:].T
    bn2 = b_node.reshape(1, h)
    w2 = W_out.reshape(1, h)
    bo2 = b_out.reshape(1, 1)

    src = edge_index[0]
    dst = edge_index[1]
    ep8 = edge_attr.reshape(e // 8, 8 * d_edge)
    bd1 = jnp.kron(jnp.eye(8, dtype=jnp.float32), wet)
    bd2 = jnp.kron(jnp.eye(8, dtype=jnp.float32), b_mat)
    be8 = jnp.tile(b_edge, 8).reshape(1, 8 * h)
    bc8 = jnp.tile(b_conv, 8).reshape(1, 8 * h)
    pm = jnp.pad(post_mask, (0, p_pad - p))
    zeros_acc = jnp.zeros((n_pad, 32), jnp.float32)

    nb = 2048
    ng = -(-n // nb)
    node_emb, tn = pl.pallas_call(
        _node_body,
        grid=(ng,),
        in_specs=[
            pl.BlockSpec((nb, d_node), lambda i: (i, 0)),
            pl.BlockSpec((d_node, h), lambda i: (0, 0)),
            pl.BlockSpec((1, h), lambda i: (0, 0)),
            pl.BlockSpec((h, h), lambda i: (0, 0)),
        ],
        out_specs=[pl.BlockSpec((nb, h), lambda i: (i, 0)),
                   pl.BlockSpec((nb, h), lambda i: (i, 0))],
        out_shape=[jax.ShapeDtypeStruct((n, h), jnp.float32),
                   jax.ShapeDtypeStruct((n, h), jnp.float32)],
    )(node_features, wnt, bn2, a_mat)

    src2 = src.reshape(total_chunks, _CB)
    dst2 = dst.reshape(total_chunks, _CB)
    partials_n = _make_tn_scatter_kernel(n_pad, total_chunks)(
        src2, dst2, tn, zeros_acc)

    eb = 8000
    eg = (e // 8) // eb
    te = pl.pallas_call(
        _edge_body,
        grid=(eg,),
        in_specs=[
            pl.BlockSpec((eb, 8 * d_edge), lambda i: (i, 0)),
            pl.BlockSpec((8 * d_edge, 8 * h), lambda i: (0, 0)),
            pl.BlockSpec((1, 8 * h), lambda i: (0, 0)),
            pl.BlockSpec((8 * h, 8 * h), lambda i: (0, 0)),
            pl.BlockSpec((1, 8 * h), lambda i: (0, 0)),
        ],
        out_specs=pl.BlockSpec((2 * eb, 4 * h), lambda i: (i, 0)),
        out_shape=jax.ShapeDtypeStruct((e // 4, 4 * h), jnp.float32),
    )(ep8, bd1, be8, bd2, bc8)

    partials = _make_te_scatter_kernel(n_pad, total_chunks)(
        dst2, te.reshape(e, h), partials_n)

    fb = 1024
    fg = -(-n_pad // fb)
    z = pl.pallas_call(
        _final_body,
        grid=(fg,),
        in_specs=[
            pl.BlockSpec((fb, h), lambda i: (i, 0)),
            pl.BlockSpec((fb, h), lambda i: (i, 0)),
            pl.BlockSpec((fb, h), lambda i: (i, 0)),
            pl.BlockSpec((1, h), lambda i: (0, 0)),
            pl.BlockSpec((1, 1), lambda i: (0, 0)),
        ],
        out_specs=pl.BlockSpec((fb, 1), lambda i: (i, 0)),
        out_shape=jax.ShapeDtypeStruct((n, 1), jnp.float32),
    )(node_emb, partials[0], partials[1], w2, bo2)

    out = _make_gather_kernel(n, p_pad)(z.reshape(n), pm)
    return out[:p]
